# baseline (speedup 1.0000x reference)
import functools
import math

import jax
import jax.numpy as jnp
from jax import lax
from jax.experimental import pallas as pl
from jax.experimental.pallas import tpu as pltpu
from jax.experimental.pallas import tpu_sc as plsc

D_MODEL = 64
ROWS = 4096
COLS = 200
NUM_CORES = 2
NUM_SUBCORES = 16
NW = NUM_CORES * NUM_SUBCORES
RPW = ROWS // NW
SPLIT = 104
NBUF = 2
SCALE = math.sqrt(D_MODEL)

_mesh = plsc.VectorSubcoreMesh(core_axis_name="c", subcore_axis_name="s")


@functools.partial(
    pl.kernel,
    mesh=_mesh,
    compiler_params=pltpu.CompilerParams(use_tc_tiling_on_sc=False),
    out_type=jax.ShapeDtypeStruct((ROWS * COLS, 2 * D_MODEL), jnp.float32),
    scratch_types=[
        pltpu.VMEM((RPW, COLS), jnp.int32),
        pltpu.VMEM((NBUF, COLS, D_MODEL), jnp.float32),
        pltpu.VMEM((NBUF, COLS, 2 * D_MODEL), jnp.float32),
        pltpu.SemaphoreType.DMA((NBUF,)),
        pltpu.SemaphoreType.DMA((NBUF,)),
    ],
)
def _emb_lookup(x_hbm, lut_hbm, out_hbm, idx_v, gbuf, sbuf, gsem, ssem):
    wid = lax.axis_index("s") * NUM_CORES + lax.axis_index("c")
    base = wid * RPW

    pltpu.sync_copy(x_hbm.at[pl.ds(base, RPW)], idx_v)

    def fire_gathers(r, b):
        pltpu.async_copy(
            lut_hbm.at[idx_v.at[r, pl.ds(0, SPLIT)]],
            gbuf.at[b, pl.ds(0, SPLIT)], gsem.at[b])
        pltpu.async_copy(
            lut_hbm.at[idx_v.at[r, pl.ds(SPLIT, COLS - SPLIT)]],
            gbuf.at[b, pl.ds(SPLIT, COLS - SPLIT)], gsem.at[b])

    def wait_gathers(r, b):
        pltpu.make_async_copy(
            lut_hbm.at[idx_v.at[r, pl.ds(0, SPLIT)]],
            gbuf.at[b, pl.ds(0, SPLIT)], gsem.at[b]).wait()
        pltpu.make_async_copy(
            lut_hbm.at[idx_v.at[r, pl.ds(SPLIT, COLS - SPLIT)]],
            gbuf.at[b, pl.ds(SPLIT, COLS - SPLIT)], gsem.at[b]).wait()

    for b in range(NBUF):
        fire_gathers(b, b)

    def row_body(it, carry):
        r0 = it * NBUF
        for b in range(NBUF):
            r = r0 + b
            wait_gathers(r, b)

            @pl.when(r >= NBUF)
            def _():
                pltpu.make_async_copy(
                    sbuf.at[b],
                    out_hbm.at[pl.ds((base + r - NBUF) * COLS, COLS)],
                    ssem.at[b]).wait()

            def scale_body(p4, c2):
                p0 = p4 * 4
                for dp in range(4):
                    for c in range(D_MODEL // 16):
                        sl = pl.ds(c * 16, 16)
                        sbuf[b, p0 + dp, sl] = gbuf[b, p0 + dp, sl] * SCALE
                return c2

            lax.fori_loop(0, COLS // 4, scale_body, 0)

            @pl.when(r + NBUF < RPW)
            def _():
                fire_gathers(r + NBUF, b)

            pltpu.async_copy(
                sbuf.at[b], out_hbm.at[pl.ds((base + r) * COLS, COLS)],
                ssem.at[b])
        return carry

    lax.fori_loop(0, RPW // NBUF, row_body, 0)

    for b in range(NBUF):
        r = RPW - NBUF + b
        pltpu.make_async_copy(
            sbuf.at[b], out_hbm.at[pl.ds((base + r) * COLS, COLS)],
            ssem.at[b]).wait()


def kernel(x, lut):
    out2 = _emb_lookup(x.astype(jnp.int32), lut)
    return out2[:, :D_MODEL].reshape(ROWS, COLS, D_MODEL)


# speedup vs baseline: 1.0174x; 1.0174x over previous
import functools
import math

import jax
import jax.numpy as jnp
from jax import lax
from jax.experimental import pallas as pl
from jax.experimental.pallas import tpu as pltpu
from jax.experimental.pallas import tpu_sc as plsc

D_MODEL = 64
ROWS = 4096
COLS = 200
NUM_CORES = 2
NUM_SUBCORES = 16
NW = NUM_CORES * NUM_SUBCORES
RPW = ROWS // NW
SPLIT = 104
NBUF = 4
SCALE = math.sqrt(D_MODEL)

_mesh = plsc.VectorSubcoreMesh(core_axis_name="c", subcore_axis_name="s")


@functools.partial(
    pl.kernel,
    mesh=_mesh,
    compiler_params=pltpu.CompilerParams(use_tc_tiling_on_sc=False),
    out_type=jax.ShapeDtypeStruct((ROWS, COLS, D_MODEL), jnp.float32),
    scratch_types=[
        pltpu.VMEM((RPW, COLS), jnp.int32),
        pltpu.VMEM((NBUF, COLS, D_MODEL), jnp.float32),
        pltpu.VMEM((NBUF, COLS, D_MODEL), jnp.float32),
        pltpu.SemaphoreType.DMA((NBUF,)),
        pltpu.SemaphoreType.DMA((NBUF,)),
    ],
)
def _emb_lookup(x_hbm, lut_hbm, out_hbm, idx_v, gbuf, sbuf, gsem, ssem):
    wid = lax.axis_index("s") * NUM_CORES + lax.axis_index("c")
    base = wid * RPW

    pltpu.sync_copy(x_hbm.at[pl.ds(base, RPW)], idx_v)

    def fire_gathers(r, b):
        pltpu.async_copy(
            lut_hbm.at[idx_v.at[r, pl.ds(0, SPLIT)]],
            gbuf.at[b, pl.ds(0, SPLIT)], gsem.at[b])
        pltpu.async_copy(
            lut_hbm.at[idx_v.at[r, pl.ds(SPLIT, COLS - SPLIT)]],
            gbuf.at[b, pl.ds(SPLIT, COLS - SPLIT)], gsem.at[b])

    def wait_gathers(r, b):
        pltpu.make_async_copy(
            lut_hbm.at[idx_v.at[r, pl.ds(0, SPLIT)]],
            gbuf.at[b, pl.ds(0, SPLIT)], gsem.at[b]).wait()
        pltpu.make_async_copy(
            lut_hbm.at[idx_v.at[r, pl.ds(SPLIT, COLS - SPLIT)]],
            gbuf.at[b, pl.ds(SPLIT, COLS - SPLIT)], gsem.at[b]).wait()

    for b in range(NBUF):
        fire_gathers(b, b)

    def row_body(it, carry):
        r0 = it * NBUF
        for b in range(NBUF):
            r = r0 + b
            wait_gathers(r, b)

            @pl.when(r >= NBUF)
            def _():
                pltpu.make_async_copy(
                    sbuf.at[b], out_hbm.at[base + r - NBUF],
                    ssem.at[b]).wait()

            def scale_body(p4, c2):
                p0 = p4 * 4
                for dp in range(4):
                    for c in range(D_MODEL // 16):
                        sl = pl.ds(c * 16, 16)
                        sbuf[b, p0 + dp, sl] = gbuf[b, p0 + dp, sl] * SCALE
                return c2

            lax.fori_loop(0, COLS // 4, scale_body, 0)

            @pl.when(r + NBUF < RPW)
            def _():
                fire_gathers(r + NBUF, b)

            pltpu.async_copy(sbuf.at[b], out_hbm.at[base + r], ssem.at[b])
        return carry

    lax.fori_loop(0, RPW // NBUF, row_body, 0)

    for b in range(NBUF):
        r = RPW - NBUF + b
        pltpu.make_async_copy(
            sbuf.at[b], out_hbm.at[base + r], ssem.at[b]).wait()


def kernel(x, lut):
    return _emb_lookup(x.astype(jnp.int32), lut)
